# baseline (speedup 1.0000x reference)
import functools

import jax
import jax.numpy as jnp
from jax import lax
from jax.experimental import pallas as pl
from jax.experimental.pallas import tpu as pltpu
from jax.experimental.pallas import tpu_sc as plsc

D = 128
NC, NS = 2, 16
NW = NC * NS

N_NODES_, NAF_ = 10000, 9
N_EDGES_, NBF_ = 320000, 3
ATOM_V, BOND_V = 1152, 384
ATOM_B = N_NODES_ * NAF_
BOND_B = N_EDGES_ * NBF_

CHA = 312
ATOM_TAIL = NW * CHA

RB = 768


def _sc_atom(x_flat, atom_table):
  mesh = plsc.VectorSubcoreMesh(core_axis_name="c", subcore_axis_name="s")

  @functools.partial(
      pl.kernel,
      out_type=jax.ShapeDtypeStruct((ATOM_B, D), jnp.float32),
      mesh=mesh,
      scratch_types=[
          pltpu.VMEM_SHARED((ATOM_V, D), jnp.float32),
          pltpu.VMEM((CHA,), jnp.int32),
          pltpu.VMEM((CHA,), jnp.int32),
          pltpu.VMEM((16,), jnp.int32),
          pltpu.VMEM((CHA, D), jnp.float32),
          pltpu.VMEM((CHA, D), jnp.float32),
          pltpu.VMEM((16, D), jnp.float32),
          pltpu.SemaphoreType.DMA,
          pltpu.SemaphoreType.DMA,
          pltpu.SemaphoreType.DMA,
          pltpu.SemaphoreType.DMA,
      ],
  )
  def k(x_hbm, at_hbm, aout_hbm, at_sp, iA, iB, ti, rA, rB_, tr,
        gsA, gsB, ssA, ssB):
    sid = lax.axis_index("s")
    wid = sid * NC + lax.axis_index("c")

    @pl.when(sid == 0)
    def _():
      pltpu.sync_copy(at_hbm, at_sp)

    plsc.subcore_barrier()

    row0 = lambda c: c * N_NODES_ + wid * CHA

    def start_gather(ibuf, rbuf, sem, c):
      pltpu.sync_copy(x_hbm.at[pl.ds(row0(c), CHA)], ibuf)
      return pltpu.async_copy(at_sp.at[ibuf], rbuf, sem)

    def start_scatter(rbuf, sem, c):
      return pltpu.async_copy(rbuf, aout_hbm.at[pl.ds(row0(c), CHA)], sem)

    def drain_scatter(rbuf, sem, c):
      pltpu.make_async_copy(rbuf, aout_hbm.at[pl.ds(row0(c), CHA)], sem).wait()

    def pair(t, carry):
      c0 = 2 * t
      c1 = c0 + 1
      @pl.when(t > 0)
      def _():
        drain_scatter(rA, ssA, c0)
        drain_scatter(rB_, ssB, c1)

      gA = start_gather(iA, rA, gsA, c0)
      gB = start_gather(iB, rB_, gsB, c1)
      gA.wait()
      start_scatter(rA, ssA, c0)
      gB.wait()
      start_scatter(rB_, ssB, c1)
      return carry

    lax.fori_loop(0, NAF_ // 2, pair, 0)
    drain_scatter(rA, ssA, 0)
    drain_scatter(rB_, ssB, 0)
    c = NAF_ - 1
    g = start_gather(iA, rA, gsA, c)
    g.wait()
    start_scatter(rA, ssA, c)
    drain_scatter(rA, ssA, c)

    for f in range(NAF_):
      @pl.when(wid == f)
      def _(f=f):
        base = f * N_NODES_ + ATOM_TAIL
        pltpu.sync_copy(x_hbm.at[pl.ds(base, 16)], ti)
        pltpu.async_copy(at_sp.at[ti], tr, gsA).wait()
        pltpu.async_copy(tr, aout_hbm.at[pl.ds(base, 16)], ssA).wait()

  return k(x_flat, atom_table)


def _tc_bond_kernel(idx_ref, tab_ref, out_ref):
  idx = idx_ref[0, 0, :]
  onehot = (idx[:, None] ==
            lax.broadcasted_iota(jnp.int32, (RB, BOND_V), 1)
            ).astype(jnp.bfloat16)
  out_ref[...] = jnp.dot(onehot, tab_ref[...],
                         preferred_element_type=jnp.float32)


def _tc_bond(e_flat, bond_table):
  nb = BOND_B // RB
  assert BOND_B % RB == 0
  tab16 = bond_table.astype(jnp.bfloat16)
  e3 = e_flat.reshape(nb, 1, RB)
  return pl.pallas_call(
      _tc_bond_kernel,
      grid=(nb,),
      in_specs=[
          pl.BlockSpec((1, 1, RB), lambda i: (i, 0, 0)),
          pl.BlockSpec((BOND_V, D), lambda i: (0, 0)),
      ],
      out_specs=pl.BlockSpec((RB, D), lambda i: (i, 0)),
      out_shape=jax.ShapeDtypeStruct((BOND_B, D), jnp.float32),
      compiler_params=pltpu.CompilerParams(
          dimension_semantics=("arbitrary",)),
  )(e3, tab16)


def kernel(x, edge_attr, atom_table, bond_table):
  x_flat = x.T.reshape(ATOM_B)
  e_flat = edge_attr.T.reshape(BOND_B)
  a_out = _sc_atom(x_flat, atom_table)
  b_out = _tc_bond(e_flat, bond_table)
  return (
      a_out.reshape(NAF_, N_NODES_, D).transpose(1, 0, 2),
      b_out.reshape(NBF_, N_EDGES_, D).transpose(1, 0, 2),
  )


# speedup vs baseline: 2.5715x; 2.5715x over previous
import functools

import jax
import jax.numpy as jnp
from jax import lax
from jax.experimental import pallas as pl
from jax.experimental.pallas import tpu as pltpu
from jax.experimental.pallas import tpu_sc as plsc

D = 128
NC, NS = 2, 16
NW = NC * NS

N_NODES_, NAF_ = 10000, 9
N_EDGES_, NBF_ = 320000, 3
ATOM_V, BOND_V = 1152, 384
ATOM_B = N_NODES_ * NAF_
BOND_B = N_EDGES_ * NBF_

CHB = 400
BOND_PW = N_EDGES_ // NW
BOND_NCH = BOND_PW // CHB

RBA = 720


def _sc_bond(e_flat, bond_table):
  mesh = plsc.VectorSubcoreMesh(core_axis_name="c", subcore_axis_name="s")

  @functools.partial(
      pl.kernel,
      out_type=jax.ShapeDtypeStruct((BOND_B, D), jnp.float32),
      mesh=mesh,
      scratch_types=[
          pltpu.VMEM_SHARED((BOND_V, D), jnp.float32),
          pltpu.VMEM((BOND_PW,), jnp.int32),
          pltpu.VMEM((CHB, D), jnp.float32),
          pltpu.VMEM((CHB, D), jnp.float32),
          pltpu.SemaphoreType.DMA,
          pltpu.SemaphoreType.DMA,
          pltpu.SemaphoreType.DMA,
          pltpu.SemaphoreType.DMA,
      ],
  )
  def k(e_hbm, bt_hbm, bout_hbm, bt_sp, bidx, rA, rB, gsA, gsB, ssA, ssB):
    sid = lax.axis_index("s")
    wid = sid * NC + lax.axis_index("c")

    @pl.when(sid == 0)
    def _():
      pltpu.sync_copy(bt_hbm, bt_sp)

    plsc.subcore_barrier()

    for f in range(NBF_):
      pbase = f * N_EDGES_ + wid * BOND_PW
      pltpu.sync_copy(e_hbm.at[pl.ds(pbase, BOND_PW)], bidx)

      def start_gather(rbuf, sem, c):
        return pltpu.async_copy(
            bt_sp.at[bidx.at[pl.ds(c * CHB, CHB)]], rbuf, sem)

      def start_scatter(rbuf, sem, c, pbase=pbase):
        return pltpu.async_copy(
            rbuf, bout_hbm.at[pl.ds(pbase + c * CHB, CHB)], sem)

      def drain_scatter(rbuf, sem, c, pbase=pbase):
        pltpu.make_async_copy(
            rbuf, bout_hbm.at[pl.ds(pbase + c * CHB, CHB)], sem).wait()

      def pair(t, carry):
        c0 = 2 * t
        c1 = c0 + 1
        @pl.when(t > 0)
        def _():
          drain_scatter(rA, ssA, c0)
          drain_scatter(rB, ssB, c1)

        gA = start_gather(rA, gsA, c0)
        gB = start_gather(rB, gsB, c1)
        gA.wait()
        start_scatter(rA, ssA, c0)
        gB.wait()
        start_scatter(rB, ssB, c1)
        return carry

      lax.fori_loop(0, BOND_NCH // 2, pair, 0)
      drain_scatter(rA, ssA, 0)
      drain_scatter(rB, ssB, 0)
      c = BOND_NCH - 1
      g = start_gather(rA, gsA, c)
      g.wait()
      start_scatter(rA, ssA, c)
      drain_scatter(rA, ssA, c)

  return k(e_flat, bond_table)


def _tc_atom_kernel(idx_ref, tab_ref, out_ref):
  idx = idx_ref[0, 0, :]
  onehot = (idx[:, None] ==
            lax.broadcasted_iota(jnp.int32, (RBA, ATOM_V), 1)
            ).astype(jnp.bfloat16)
  out_ref[...] = jnp.dot(onehot, tab_ref[...],
                         preferred_element_type=jnp.float32)


def _tc_atom(x_flat, atom_table):
  nb = ATOM_B // RBA
  tab16 = atom_table.astype(jnp.bfloat16)
  x3 = x_flat.reshape(nb, 1, RBA)
  return pl.pallas_call(
      _tc_atom_kernel,
      grid=(nb,),
      in_specs=[
          pl.BlockSpec((1, 1, RBA), lambda i: (i, 0, 0)),
          pl.BlockSpec((ATOM_V, D), lambda i: (0, 0)),
      ],
      out_specs=pl.BlockSpec((RBA, D), lambda i: (i, 0)),
      out_shape=jax.ShapeDtypeStruct((ATOM_B, D), jnp.float32),
      compiler_params=pltpu.CompilerParams(
          dimension_semantics=("arbitrary",)),
  )(x3, tab16)


def kernel(x, edge_attr, atom_table, bond_table):
  x_flat = x.T.reshape(ATOM_B)
  e_flat = edge_attr.T.reshape(BOND_B)
  a_out = _tc_atom(x_flat, atom_table)
  b_out = _sc_bond(e_flat, bond_table)
  return (
      a_out.reshape(NAF_, N_NODES_, D).transpose(1, 0, 2),
      b_out.reshape(NBF_, N_EDGES_, D).transpose(1, 0, 2),
  )


# speedup vs baseline: 2.5956x; 1.0094x over previous
import functools

import jax
import jax.numpy as jnp
from jax import lax
from jax.experimental import pallas as pl
from jax.experimental.pallas import tpu as pltpu
from jax.experimental.pallas import tpu_sc as plsc

D = 128
NC, NS = 2, 16
NW = NC * NS

N_NODES_, NAF_ = 10000, 9
N_EDGES_, NBF_ = 320000, 3
ATOM_V, BOND_V = 1152, 384
ATOM_B = N_NODES_ * NAF_
BOND_B = N_EDGES_ * NBF_

CHB = 200
BOND_PW = N_EDGES_ // NW
BOND_NCH = BOND_PW // CHB

RBA = 720


def _sc_bond(e_flat, bond_table):
  mesh = plsc.VectorSubcoreMesh(core_axis_name="c", subcore_axis_name="s")

  @functools.partial(
      pl.kernel,
      out_type=jax.ShapeDtypeStruct((BOND_B, D), jnp.float32),
      mesh=mesh,
      scratch_types=[
          pltpu.VMEM_SHARED((BOND_V, D), jnp.float32),
          pltpu.VMEM((BOND_PW,), jnp.int32),
          pltpu.VMEM((CHB, D), jnp.float32),
          pltpu.VMEM((CHB, D), jnp.float32),
          pltpu.SemaphoreType.DMA,
          pltpu.SemaphoreType.DMA,
          pltpu.SemaphoreType.DMA,
          pltpu.SemaphoreType.DMA,
      ],
  )
  def k(e_hbm, bt_hbm, bout_hbm, bt_sp, bidx, rA, rB, gsA, gsB, ssA, ssB):
    sid = lax.axis_index("s")
    wid = sid * NC + lax.axis_index("c")

    @pl.when(sid == 0)
    def _():
      pltpu.sync_copy(bt_hbm, bt_sp)

    plsc.subcore_barrier()

    for f in range(NBF_):
      pbase = f * N_EDGES_ + wid * BOND_PW
      pltpu.sync_copy(e_hbm.at[pl.ds(pbase, BOND_PW)], bidx)

      def start_gather(rbuf, sem, c):
        return pltpu.async_copy(
            bt_sp.at[bidx.at[pl.ds(c * CHB, CHB)]], rbuf, sem)

      def start_scatter(rbuf, sem, c, pbase=pbase):
        return pltpu.async_copy(
            rbuf, bout_hbm.at[pl.ds(pbase + c * CHB, CHB)], sem)

      def drain_scatter(rbuf, sem, c, pbase=pbase):
        pltpu.make_async_copy(
            rbuf, bout_hbm.at[pl.ds(pbase + c * CHB, CHB)], sem).wait()

      def pair(t, carry, f=f):
        c0 = 2 * t
        c1 = c0 + 1
        if f == 0:
          @pl.when(t > 0)
          def _():
            drain_scatter(rA, ssA, c0)
            drain_scatter(rB, ssB, c1)
        else:
          drain_scatter(rA, ssA, c0)
          drain_scatter(rB, ssB, c1)

        gA = start_gather(rA, gsA, c0)
        gB = start_gather(rB, gsB, c1)
        gA.wait()
        start_scatter(rA, ssA, c0)
        gB.wait()
        start_scatter(rB, ssB, c1)
        return carry

      lax.fori_loop(0, BOND_NCH // 2, pair, 0)

    final_base = (NBF_ - 1) * N_EDGES_ + wid * BOND_PW
    pltpu.make_async_copy(
        rA, bout_hbm.at[pl.ds(final_base, CHB)], ssA).wait()
    pltpu.make_async_copy(
        rB, bout_hbm.at[pl.ds(final_base, CHB)], ssB).wait()

  return k(e_flat, bond_table)


def _tc_atom_kernel(idx_ref, tab_ref, out_ref):
  idx = idx_ref[0, 0, :]
  onehot = (idx[:, None] ==
            lax.broadcasted_iota(jnp.int32, (RBA, ATOM_V), 1)
            ).astype(jnp.bfloat16)
  out_ref[...] = jnp.dot(onehot, tab_ref[...],
                         preferred_element_type=jnp.float32)


def _tc_atom(x_flat, atom_table):
  nb = ATOM_B // RBA
  tab16 = atom_table.astype(jnp.bfloat16)
  x3 = x_flat.reshape(nb, 1, RBA)
  return pl.pallas_call(
      _tc_atom_kernel,
      grid=(nb,),
      in_specs=[
          pl.BlockSpec((1, 1, RBA), lambda i: (i, 0, 0)),
          pl.BlockSpec((ATOM_V, D), lambda i: (0, 0)),
      ],
      out_specs=pl.BlockSpec((RBA, D), lambda i: (i, 0)),
      out_shape=jax.ShapeDtypeStruct((ATOM_B, D), jnp.float32),
      compiler_params=pltpu.CompilerParams(
          dimension_semantics=("arbitrary",)),
  )(x3, tab16)


def kernel(x, edge_attr, atom_table, bond_table):
  x_flat = x.T.reshape(ATOM_B)
  e_flat = edge_attr.T.reshape(BOND_B)
  a_out = _tc_atom(x_flat, atom_table)
  b_out = _sc_bond(e_flat, bond_table)
  return (
      a_out.reshape(NAF_, N_NODES_, D).transpose(1, 0, 2),
      b_out.reshape(NBF_, N_EDGES_, D).transpose(1, 0, 2),
  )


# speedup vs baseline: 3.7510x; 1.4451x over previous
import functools

import jax
import jax.numpy as jnp
from jax import lax
from jax.experimental import pallas as pl
from jax.experimental.pallas import tpu as pltpu
from jax.experimental.pallas import tpu_sc as plsc

D = 128
NC, NS = 2, 16
NW = NC * NS

N_NODES_, NAF_ = 10000, 9
N_EDGES_, NBF_ = 320000, 3
ATOM_V, BOND_V = 1152, 384
ATOM_B = N_NODES_ * NAF_
BOND_B = N_EDGES_ * NBF_

CHB = 200
BOND_PW = N_EDGES_ // NW
BOND_NCH = BOND_PW // CHB

RBA = 720


def _sc_bond(e_flat, bond_table):
  mesh = plsc.VectorSubcoreMesh(core_axis_name="c", subcore_axis_name="s")

  @functools.partial(
      pl.kernel,
      out_type=jax.ShapeDtypeStruct((BOND_B, D), jnp.float32),
      mesh=mesh,
      scratch_types=[
          pltpu.VMEM_SHARED((BOND_V, D), jnp.float32),
          pltpu.VMEM((BOND_PW,), jnp.int32),
          pltpu.VMEM((CHB, D), jnp.float32),
          pltpu.VMEM((CHB, D), jnp.float32),
          pltpu.VMEM((CHB, D), jnp.float32),
          pltpu.VMEM((CHB, D), jnp.float32),
          pltpu.SemaphoreType.DMA,
          pltpu.SemaphoreType.DMA,
          pltpu.SemaphoreType.DMA,
          pltpu.SemaphoreType.DMA,
          pltpu.SemaphoreType.DMA,
          pltpu.SemaphoreType.DMA,
          pltpu.SemaphoreType.DMA,
          pltpu.SemaphoreType.DMA,
      ],
  )
  def k(e_hbm, bt_hbm, bout_hbm, bt_sp, bidx, rA, rB, rC, rD,
        gsA, gsB, gsC, gsD, ssA, ssB, ssC, ssD):
    sid = lax.axis_index("s")
    wid = sid * NC + lax.axis_index("c")

    @pl.when(sid == 0)
    def _():
      pltpu.sync_copy(bt_hbm, bt_sp)

    plsc.subcore_barrier()

    pair_sets = ((rA, gsA, ssA, rB, gsB, ssB), (rC, gsC, ssC, rD, gsD, ssD))
    pairs_per_plane = BOND_NCH // 2

    for f in range(NBF_):
      pbase = f * N_EDGES_ + wid * BOND_PW
      pltpu.sync_copy(e_hbm.at[pl.ds(pbase, BOND_PW)], bidx)

      def start_gather(rbuf, sem, c):
        return pltpu.async_copy(
            bt_sp.at[bidx.at[pl.ds(c * CHB, CHB)]], rbuf, sem)

      def start_scatter(rbuf, sem, c, pbase=pbase):
        return pltpu.async_copy(
            rbuf, bout_hbm.at[pl.ds(pbase + c * CHB, CHB)], sem)

      def drain_scatter(rbuf, sem, c, pbase=pbase):
        pltpu.make_async_copy(
            rbuf, bout_hbm.at[pl.ds(pbase + c * CHB, CHB)], sem).wait()

      def pair(t, carry, f=f):
        c0 = 2 * t
        c1 = c0 + 1

        def run_set(s, skip_drain_below, t=t, c0=c0, c1=c1):
          bA, gA_s, sA_s, bB, gB_s, sB_s = pair_sets[s]
          if skip_drain_below is None:
            drain_scatter(bA, sA_s, c0)
            drain_scatter(bB, sB_s, c1)
          else:
            @pl.when(t > skip_drain_below)
            def _():
              drain_scatter(bA, sA_s, c0)
              drain_scatter(bB, sB_s, c1)
          gA = start_gather(bA, gA_s, c0)
          gB = start_gather(bB, gB_s, c1)
          gA.wait()
          start_scatter(bA, sA_s, c0)
          gB.wait()
          start_scatter(bB, sB_s, c1)

        even_set = f % 2
        @pl.when(lax.rem(t, 2) == 0)
        def _():
          run_set(even_set, (1 if f == 0 else None))
        @pl.when(lax.rem(t, 2) == 1)
        def _():
          run_set(1 - even_set, (2 if f == 0 else None))
        return carry

      lax.fori_loop(0, pairs_per_plane, pair, 0)

    final_base = (NBF_ - 1) * N_EDGES_ + wid * BOND_PW
    for rbuf, sem in ((rA, ssA), (rB, ssB), (rC, ssC), (rD, ssD)):
      pltpu.make_async_copy(
          rbuf, bout_hbm.at[pl.ds(final_base, CHB)], sem).wait()

  return k(e_flat, bond_table)


def _tc_atom_kernel(idx_ref, tab_ref, out_ref):
  idx = idx_ref[0, 0, :]
  onehot = (idx[:, None] ==
            lax.broadcasted_iota(jnp.int32, (RBA, ATOM_V), 1)
            ).astype(jnp.bfloat16)
  out_ref[...] = jnp.dot(onehot, tab_ref[...],
                         preferred_element_type=jnp.float32)


def _tc_atom(x_flat, atom_table):
  nb = ATOM_B // RBA
  tab16 = atom_table.astype(jnp.bfloat16)
  x3 = x_flat.reshape(nb, 1, RBA)
  return pl.pallas_call(
      _tc_atom_kernel,
      grid=(nb,),
      in_specs=[
          pl.BlockSpec((1, 1, RBA), lambda i: (i, 0, 0)),
          pl.BlockSpec((ATOM_V, D), lambda i: (0, 0)),
      ],
      out_specs=pl.BlockSpec((RBA, D), lambda i: (i, 0)),
      out_shape=jax.ShapeDtypeStruct((ATOM_B, D), jnp.float32),
      compiler_params=pltpu.CompilerParams(
          dimension_semantics=("arbitrary",)),
  )(x3, tab16)


def kernel(x, edge_attr, atom_table, bond_table):
  x_flat = x.T.reshape(ATOM_B)
  e_flat = edge_attr.T.reshape(BOND_B)
  a_out = _tc_atom(x_flat, atom_table)
  b_out = _sc_bond(e_flat, bond_table)
  return (
      a_out.reshape(NAF_, N_NODES_, D).transpose(1, 0, 2),
      b_out.reshape(NBF_, N_EDGES_, D).transpose(1, 0, 2),
  )


# speedup vs baseline: 3.8014x; 1.0134x over previous
import functools

import jax
import jax.numpy as jnp
from jax import lax
from jax.experimental import pallas as pl
from jax.experimental.pallas import tpu as pltpu
from jax.experimental.pallas import tpu_sc as plsc

D = 128
NC, NS = 2, 16
NW = NC * NS

N_NODES_, NAF_ = 10000, 9
N_EDGES_, NBF_ = 320000, 3
ATOM_V, BOND_V = 1152, 384
ATOM_B = N_NODES_ * NAF_
BOND_B = N_EDGES_ * NBF_

CHB = 200
BOND_PW = N_EDGES_ // NW
BOND_NCH = BOND_PW // CHB

RBA = 720


def _sc_bond(e_flat, bond_table):
  mesh = plsc.VectorSubcoreMesh(core_axis_name="c", subcore_axis_name="s")

  @functools.partial(
      pl.kernel,
      out_type=jax.ShapeDtypeStruct((BOND_B, D), jnp.float32),
      mesh=mesh,
      scratch_types=[
          pltpu.VMEM_SHARED((BOND_V, D), jnp.float32),
          pltpu.VMEM((BOND_PW,), jnp.int32),
          pltpu.VMEM((CHB, D), jnp.float32),
          pltpu.VMEM((CHB, D), jnp.float32),
          pltpu.VMEM((CHB, D), jnp.float32),
          pltpu.VMEM((CHB, D), jnp.float32),
          pltpu.SemaphoreType.DMA,
          pltpu.SemaphoreType.DMA,
          pltpu.SemaphoreType.DMA,
          pltpu.SemaphoreType.DMA,
          pltpu.SemaphoreType.DMA,
          pltpu.SemaphoreType.DMA,
          pltpu.SemaphoreType.DMA,
          pltpu.SemaphoreType.DMA,
      ],
  )
  def k(e_hbm, bt_hbm, bout_hbm, bt_sp, bidx, rA, rB, rC, rD,
        gsA, gsB, gsC, gsD, ssA, ssB, ssC, ssD):
    sid = lax.axis_index("s")
    wid = sid * NC + lax.axis_index("c")

    @pl.when(sid == 0)
    def _():
      pltpu.sync_copy(bt_hbm, bt_sp)

    plsc.subcore_barrier()

    pair_sets = ((rA, gsA, ssA, rB, gsB, ssB), (rC, gsC, ssC, rD, gsD, ssD))
    pairs_per_plane = BOND_NCH // 2

    for f in range(NBF_):
      pbase = f * N_EDGES_ + wid * BOND_PW
      pltpu.sync_copy(e_hbm.at[pl.ds(pbase, BOND_PW)], bidx)

      def start_gather(rbuf, sem, c):
        return pltpu.async_copy(
            bt_sp.at[bidx.at[pl.ds(c * CHB, CHB)]], rbuf, sem)

      def start_scatter(rbuf, sem, c, pbase=pbase):
        return pltpu.async_copy(
            rbuf, bout_hbm.at[pl.ds(pbase + c * CHB, CHB)], sem)

      def drain_scatter(rbuf, sem, c, pbase=pbase):
        pltpu.make_async_copy(
            rbuf, bout_hbm.at[pl.ds(pbase + c * CHB, CHB)], sem).wait()

      def wait_gather(rbuf, sem, c):
        pltpu.make_async_copy(
            bt_sp.at[bidx.at[pl.ds(c * CHB, CHB)]], rbuf, sem).wait()

      def pair(t, carry, f=f):
        c0 = 2 * t
        c1 = c0 + 1

        def issue_set(s, skip_drain_below, t=t, c0=c0, c1=c1):
          bA, gA_s, sA_s, bB, gB_s, sB_s = pair_sets[s]
          if skip_drain_below is None:
            drain_scatter(bA, sA_s, c0)
            drain_scatter(bB, sB_s, c1)
          else:
            @pl.when(t > skip_drain_below)
            def _():
              drain_scatter(bA, sA_s, c0)
              drain_scatter(bB, sB_s, c1)
          start_gather(bA, gA_s, c0)
          start_gather(bB, gB_s, c1)

        def scatter_prev_set(s, t=t, c0=c0, c1=c1):
          bA, gA_s, sA_s, bB, gB_s, sB_s = pair_sets[s]
          wait_gather(bA, gA_s, c0 - 2)
          start_scatter(bA, sA_s, c0 - 2)
          wait_gather(bB, gB_s, c1 - 2)
          start_scatter(bB, sB_s, c1 - 2)

        even_set = f % 2
        @pl.when(lax.rem(t, 2) == 0)
        def _():
          issue_set(even_set, (1 if f == 0 else None))
          @pl.when(t > 0)
          def _():
            scatter_prev_set(1 - even_set)
        @pl.when(lax.rem(t, 2) == 1)
        def _():
          issue_set(1 - even_set, (2 if f == 0 else None))
          scatter_prev_set(even_set)
        return carry

      lax.fori_loop(0, pairs_per_plane, pair, 0)

      last_set = f % 2
      bA, gA_s, sA_s, bB, gB_s, sB_s = pair_sets[last_set]
      cl0 = BOND_NCH - 2
      wait_gather(bA, gA_s, cl0)
      start_scatter(bA, sA_s, cl0)
      wait_gather(bB, gB_s, cl0 + 1)
      start_scatter(bB, sB_s, cl0 + 1)

    final_base = (NBF_ - 1) * N_EDGES_ + wid * BOND_PW
    for rbuf, sem in ((rA, ssA), (rB, ssB), (rC, ssC), (rD, ssD)):
      pltpu.make_async_copy(
          rbuf, bout_hbm.at[pl.ds(final_base, CHB)], sem).wait()

  return k(e_flat, bond_table)


def _tc_atom_kernel(idx_ref, tab_ref, out_ref):
  idx = idx_ref[0, 0, :]
  onehot = (idx[:, None] ==
            lax.broadcasted_iota(jnp.int32, (RBA, ATOM_V), 1)
            ).astype(jnp.bfloat16)
  out_ref[...] = jnp.dot(onehot, tab_ref[...],
                         preferred_element_type=jnp.float32)


def _tc_atom(x_flat, atom_table):
  nb = ATOM_B // RBA
  tab16 = atom_table.astype(jnp.bfloat16)
  x3 = x_flat.reshape(nb, 1, RBA)
  return pl.pallas_call(
      _tc_atom_kernel,
      grid=(nb,),
      in_specs=[
          pl.BlockSpec((1, 1, RBA), lambda i: (i, 0, 0)),
          pl.BlockSpec((ATOM_V, D), lambda i: (0, 0)),
      ],
      out_specs=pl.BlockSpec((RBA, D), lambda i: (i, 0)),
      out_shape=jax.ShapeDtypeStruct((ATOM_B, D), jnp.float32),
      compiler_params=pltpu.CompilerParams(
          dimension_semantics=("arbitrary",)),
  )(x3, tab16)


def kernel(x, edge_attr, atom_table, bond_table):
  x_flat = x.T.reshape(ATOM_B)
  e_flat = edge_attr.T.reshape(BOND_B)
  a_out = _tc_atom(x_flat, atom_table)
  b_out = _sc_bond(e_flat, bond_table)
  return (
      a_out.reshape(NAF_, N_NODES_, D).transpose(1, 0, 2),
      b_out.reshape(NBF_, N_EDGES_, D).transpose(1, 0, 2),
  )
